# Initial kernel scaffold; baseline (speedup 1.0000x reference)
import jax
import jax.numpy as jnp
from jax.experimental import pallas as pl


def kernel(input_ids, emb_table, W, b):
    raise NotImplementedError("write your pallas kernel here")


# speedup vs baseline: 3.3125x; 3.3125x over previous
import functools

import jax
import jax.numpy as jnp
from jax import lax
from jax.experimental import pallas as pl
from jax.experimental.pallas import tpu as pltpu
from jax.experimental.pallas import tpu_sc as plsc


def _proj_body(emb_ref, w_ref, b_ref, out_ref):
    out_ref[...] = lax.dot_general(
        emb_ref[...], w_ref[...],
        dimension_numbers=(((1,), (1,)), ((), ())),
        preferred_element_type=jnp.float32,
    ) + b_ref[...]


def _project(emb, w, b_vec):
    v, o = emb.shape[0], w.shape[0]
    return pl.pallas_call(
        _proj_body,
        out_shape=jax.ShapeDtypeStruct((v, o), jnp.float32),
    )(emb, w, b_vec.reshape(1, o))


@functools.lru_cache(maxsize=None)
def _make_gather(D, B):
    info = plsc.get_sparse_core_info()
    nc, ns = info.num_cores, info.num_subcores
    nw = nc * ns
    b_per_w = B // nw
    ch = 128
    n_chunks = b_per_w // ch
    mesh = plsc.VectorSubcoreMesh(core_axis_name="c", subcore_axis_name="s")

    @functools.partial(
        pl.kernel,
        mesh=mesh,
        out_type=jax.ShapeDtypeStruct((B, D), jnp.float32),
        scratch_types=[
            pltpu.VMEM((ch,), jnp.int32),
            pltpu.VMEM((ch, D), jnp.float32),
            pltpu.SemaphoreType.DMA,
        ],
    )
    def gather(table_hbm, idx_hbm, out_hbm, idx_v, rows_v, sem):
        wid = lax.axis_index("s") * nc + lax.axis_index("c")
        base = wid * b_per_w

        def body(i, carry):
            off = pl.multiple_of(base + i * ch, 8)
            pltpu.sync_copy(idx_hbm.at[pl.ds(off, ch)], idx_v)
            pltpu.async_copy(table_hbm.at[idx_v], rows_v, sem).wait()
            pltpu.sync_copy(rows_v, out_hbm.at[pl.ds(off, ch)])
            return carry

        lax.fori_loop(0, n_chunks, body, 0)

    return gather


def kernel(input_ids, emb_table, W, b):
    bsz, seq = input_ids.shape
    o = W.shape[0]
    projected = _project(emb_table, W, b)
    idx = input_ids.reshape(-1).astype(jnp.int32)
    out = _make_gather(o, bsz * seq)(projected, idx)
    return out.reshape(bsz, seq, o)


# speedup vs baseline: 3.6699x; 1.1079x over previous
import functools

import jax
import jax.numpy as jnp
from jax import lax
from jax.experimental import pallas as pl
from jax.experimental.pallas import tpu as pltpu
from jax.experimental.pallas import tpu_sc as plsc


def _proj_body(emb_ref, w_ref, b_ref, out_ref):
    out_ref[...] = lax.dot_general(
        emb_ref[...], w_ref[...],
        dimension_numbers=(((1,), (1,)), ((), ())),
        preferred_element_type=jnp.float32,
    ) + b_ref[...]


def _project(emb, w, b_vec):
    v, o = emb.shape[0], w.shape[0]
    return pl.pallas_call(
        _proj_body,
        out_shape=jax.ShapeDtypeStruct((v, o), jnp.float32),
    )(emb, w, b_vec.reshape(1, o))


@functools.lru_cache(maxsize=None)
def _make_gather(D, B):
    info = plsc.get_sparse_core_info()
    nc, ns = info.num_cores, info.num_subcores
    nw = nc * ns
    b_per_w = B // nw
    ch = 80
    n_chunks = b_per_w // ch
    assert n_chunks * ch == b_per_w and n_chunks % 2 == 0 and ch % 8 == 0
    mesh = plsc.VectorSubcoreMesh(core_axis_name="c", subcore_axis_name="s")

    @functools.partial(
        pl.kernel,
        mesh=mesh,
        out_type=jax.ShapeDtypeStruct((B, D), jnp.float32),
        scratch_types=[
            pltpu.VMEM((b_per_w,), jnp.int32),
            pltpu.VMEM((ch, D), jnp.float32),
            pltpu.VMEM((ch, D), jnp.float32),
            pltpu.SemaphoreType.DMA,
            pltpu.SemaphoreType.DMA,
            pltpu.SemaphoreType.DMA,
            pltpu.SemaphoreType.DMA,
        ],
    )
    def gather(table_hbm, idx_hbm, out_hbm, idx_v, rows0, rows1,
               gsem0, gsem1, wsem0, wsem1):
        wid = lax.axis_index("s") * nc + lax.axis_index("c")
        base = wid * b_per_w

        def gstart(i, buf, sem):
            pltpu.make_async_copy(
                table_hbm.at[idx_v.at[pl.ds(i * ch, ch)]], buf, sem).start()

        def gwait(buf, sem):
            pltpu.make_async_copy(
                table_hbm.at[idx_v.at[pl.ds(0, ch)]], buf, sem).wait()

        def wstart(i, buf, sem):
            off = pl.multiple_of(base + i * ch, 8)
            pltpu.make_async_copy(buf, out_hbm.at[pl.ds(off, ch)], sem).start()

        def wwait(buf, sem):
            pltpu.make_async_copy(buf, out_hbm.at[pl.ds(base, ch)], sem).wait()

        pltpu.sync_copy(idx_hbm.at[pl.ds(base, b_per_w)], idx_v)

        gstart(0, rows0, gsem0)
        gstart(1, rows1, gsem1)
        gwait(rows0, gsem0)
        wstart(0, rows0, wsem0)

        def body(j2, carry):
            i = 2 * j2 + 1
            wwait(rows0, wsem0)
            gstart(i + 1, rows0, gsem0)
            gwait(rows1, gsem1)
            wstart(i, rows1, wsem1)
            wwait(rows1, wsem1)
            gstart(i + 2, rows1, gsem1)
            gwait(rows0, gsem0)
            wstart(i + 1, rows0, wsem0)
            return carry

        lax.fori_loop(0, n_chunks // 2 - 1, body, 0)

        gwait(rows1, gsem1)
        wstart(n_chunks - 1, rows1, wsem1)
        wwait(rows0, wsem0)
        wwait(rows1, wsem1)

    return gather


def kernel(input_ids, emb_table, W, b):
    bsz, seq = input_ids.shape
    o = W.shape[0]
    projected = _project(emb_table, W, b)
    idx = input_ids.reshape(-1).astype(jnp.int32)
    out = _make_gather(o, bsz * seq)(projected, idx)
    return out.reshape(bsz, seq, o)


# speedup vs baseline: 3.6701x; 1.0001x over previous
import functools

import jax
import jax.numpy as jnp
from jax import lax
from jax.experimental import pallas as pl
from jax.experimental.pallas import tpu as pltpu
from jax.experimental.pallas import tpu_sc as plsc


def _proj_body(emb_ref, w_ref, b_ref, out_ref):
    out_ref[...] = lax.dot_general(
        emb_ref[...], w_ref[...],
        dimension_numbers=(((1,), (1,)), ((), ())),
        preferred_element_type=jnp.float32,
    ) + b_ref[...]


def _project(emb, w, b_vec):
    v, o = emb.shape[0], w.shape[0]
    return pl.pallas_call(
        _proj_body,
        out_shape=jax.ShapeDtypeStruct((v, o), jnp.float32),
    )(emb, w, b_vec.reshape(1, o))


@functools.lru_cache(maxsize=None)
def _make_gather(D, B, V):
    info = plsc.get_sparse_core_info()
    nc, ns = info.num_cores, info.num_subcores
    nw = nc * ns
    b_per_w = B // nw
    ch = 80
    n_chunks = b_per_w // ch
    assert n_chunks * ch == b_per_w and n_chunks % 2 == 0 and ch % 8 == 0
    mesh = plsc.VectorSubcoreMesh(core_axis_name="c", subcore_axis_name="s")

    @functools.partial(
        pl.kernel,
        mesh=mesh,
        out_type=jax.ShapeDtypeStruct((B, D), jnp.float32),
        scratch_types=[
            pltpu.VMEM((b_per_w,), jnp.int32),
            pltpu.VMEM((ch, D), jnp.float32),
            pltpu.VMEM((ch, D), jnp.float32),
            pltpu.SemaphoreType.DMA,
            pltpu.SemaphoreType.DMA,
            pltpu.SemaphoreType.DMA,
            pltpu.SemaphoreType.DMA,
        ],
    )
    def gather(table_hbm, idx_hbm, out_hbm, idx_v, rows0, rows1,
               gsem0, gsem1, wsem0, wsem1):
        wid = lax.axis_index("s") * nc + lax.axis_index("c")
        base = wid * b_per_w

        def gstart(i, buf, sem):
            pltpu.make_async_copy(
                table_hbm.at[idx_v.at[pl.ds(i * ch, ch)]], buf, sem).start()

        def gwait(buf, sem):
            pltpu.make_async_copy(
                table_hbm.at[idx_v.at[pl.ds(0, ch)]], buf, sem).wait()

        def wstart(i, buf, sem):
            off = pl.multiple_of(base + i * ch, 8)
            pltpu.make_async_copy(buf, out_hbm.at[pl.ds(off, ch)], sem).start()

        def wwait(buf, sem):
            pltpu.make_async_copy(buf, out_hbm.at[pl.ds(base, ch)], sem).wait()

        pltpu.sync_copy(idx_hbm.at[pl.ds(base, b_per_w)], idx_v)

        gstart(0, rows0, gsem0)
        gstart(1, rows1, gsem1)
        gwait(rows0, gsem0)
        wstart(0, rows0, wsem0)

        def body(j2, carry):
            i = 2 * j2 + 1
            wwait(rows0, wsem0)
            gstart(i + 1, rows0, gsem0)
            gwait(rows1, gsem1)
            wstart(i, rows1, wsem1)
            wwait(rows1, wsem1)
            gstart(i + 2, rows1, gsem1)
            gwait(rows0, gsem0)
            wstart(i + 1, rows0, wsem0)
            return carry

        lax.fori_loop(0, n_chunks // 2 - 1, body, 0)

        gwait(rows1, gsem1)
        wstart(n_chunks - 1, rows1, wsem1)
        wwait(rows0, wsem0)
        wwait(rows1, wsem1)

    return gather


def kernel(input_ids, emb_table, W, b):
    bsz, seq = input_ids.shape
    o = W.shape[0]
    projected = _project(emb_table, W, b)
    idx = input_ids.reshape(-1).astype(jnp.int32)
    out = _make_gather(o, bsz * seq, emb_table.shape[0])(projected, idx)
    return out.reshape(bsz, seq, o)
